# baseline (speedup 1.0000x reference)
import functools

import jax
import jax.numpy as jnp
from jax import lax
from jax.experimental import pallas as pl
from jax.experimental.pallas import tpu as pltpu
from jax.experimental.pallas import tpu_sc as plsc

_B, _N, _C = 64, 3549, 5
_TOTAL = _B * _N
_W = 896
_W3 = 768
_TC0 = 3 * _W + _W3
_TW = _N - _TC0
_LN2 = 0.6931471805599453
_SQRT2H = 1.4142135623730951


def _fast_log(p):
    bits = plsc.bitcast(p, jnp.int32)
    e = (bits >> 23) - 127
    m = (bits & 0x7FFFFF) | 0x3F800000
    f = plsc.bitcast(m, jnp.float32)
    big = f > _SQRT2H
    f = jnp.where(big, f * 0.5, f)
    e = jnp.where(big, e + 1, e)
    z = (f - 1.0) / (f + 1.0)
    z2 = z * z
    poly = 1.0 + z2 * (
        0.3333333333333333
        + z2 * (0.2 + z2 * (0.14285714285714285 + z2 * 0.1111111111111111)))
    val = e.astype(jnp.float32) * _LN2 + 2.0 * z * poly
    return jnp.where(p < 1.1754944e-38, -100.0, val)


def _cell16(x0, y0, b, carry):
    face, mse, bpos, bbg = carry
    maskf = jnp.where(y0 > 0.5, 1.0, 0.0)
    face = face + maskf
    d = b[0] - b[4]
    sq = d * d
    d = b[1] - b[5]
    sq = sq + d * d
    d = b[2] - b[6]
    sq = sq + d * d
    d = b[3] - b[7]
    sq = sq + d * d
    mse = mse + maskf * sq
    logp = _fast_log(x0)
    log1mp = _fast_log(1.0 - x0)
    bpos = bpos - maskf * (y0 * logp + (1.0 - y0) * log1mp)
    bbg = bbg + (maskf - 1.0) * log1mp
    return face, mse, bpos, bbg


def _sc_body(x_hbm, y_hbm, out_hbm, xm, ym, xtl, ytl, pv, sem):
    c = lax.axis_index("c")
    s = lax.axis_index("s")
    wid = s * 2 + c
    rt = wid // 4
    cg = wid % 4
    r0 = rt * 8

    zero = jnp.zeros((16,), jnp.float32)

    for ch in range(_C):
        for r in range(8):
            for t in range(_TW // 16):
                o = pl.ds(t * 16, 16)
                xtl[ch, r, o] = zero
                ytl[ch, r, o] = zero
            o = pl.ds(_TW - 16, 16)
            xtl[ch, r, o] = zero
            ytl[ch, r, o] = zero

    col0 = cg * _W

    copies = []

    @pl.when(cg < 3)
    def _dma_main():
        cps = []
        for ch in range(_C):
            cps.append(pltpu.make_async_copy(
                x_hbm.at[ch, pl.ds(r0, 8), pl.ds(col0, _W)], xm.at[ch], sem))
            cps.append(pltpu.make_async_copy(
                y_hbm.at[ch, pl.ds(r0, 8), pl.ds(col0, _W)], ym.at[ch], sem))
        for cp in cps:
            cp.start()
        for cp in cps:
            cp.wait()

    @pl.when(cg == 3)
    def _dma_cg3():
        cps = []
        for ch in range(_C):
            cps.append(pltpu.make_async_copy(
                x_hbm.at[ch, pl.ds(r0, 8), pl.ds(3 * _W, _W3)],
                xm.at[ch, :, pl.ds(0, _W3)], sem))
            cps.append(pltpu.make_async_copy(
                y_hbm.at[ch, pl.ds(r0, 8), pl.ds(3 * _W, _W3)],
                ym.at[ch, :, pl.ds(0, _W3)], sem))
            cps.append(pltpu.make_async_copy(
                x_hbm.at[ch, pl.ds(r0, 8), pl.ds(_TC0, _TW)], xtl.at[ch], sem))
            cps.append(pltpu.make_async_copy(
                y_hbm.at[ch, pl.ds(r0, 8), pl.ds(_TC0, _TW)], ytl.at[ch], sem))
        for cp in cps:
            cp.start()
        for cp in cps:
            cp.wait()

    ng = jnp.where(cg == 3, _W3 // 16, _W // 16)

    carry = (zero, zero, zero, zero)
    for r in range(8):
        def group(g, cr, r=r):
            o = pl.ds(g * 16, 16)
            b = [xm[1 + i, r, o] for i in range(4)]
            b += [ym[1 + i, r, o] for i in range(4)]
            return _cell16(xm[0, r, o], ym[0, r, o], b, cr)
        carry = lax.fori_loop(0, ng, group, carry)

    nfull = _TW // 16
    novl = 16 - (_TW - nfull * 16)
    wtail = jnp.where(lax.iota(jnp.int32, 16) >= novl, 1.0, 0.0)
    for r in range(8):
        for t in range(nfull):
            o = pl.ds(t * 16, 16)
            b = [xtl[1 + i, r, o] for i in range(4)]
            b += [ytl[1 + i, r, o] for i in range(4)]
            carry = _cell16(xtl[0, r, o], ytl[0, r, o], b, carry)
        o = pl.ds(_TW - 16, 16)
        b = [xtl[1 + i, r, o] * wtail for i in range(4)]
        b += [ytl[1 + i, r, o] * wtail for i in range(4)]
        carry = _cell16(xtl[0, r, o] * wtail, ytl[0, r, o] * wtail, b, carry)

    face, mse, bpos, bbg = carry
    pv[pl.ds(0, 16)] = face
    pv[pl.ds(16, 16)] = mse
    pv[pl.ds(32, 16)] = bpos
    pv[pl.ds(48, 16)] = bbg
    pltpu.sync_copy(pv, out_hbm.at[pl.ds(wid * 64, 64)])


_sc_call = pl.kernel(
    _sc_body,
    out_type=jax.ShapeDtypeStruct((32 * 64,), jnp.float32),
    mesh=plsc.VectorSubcoreMesh(core_axis_name="c", subcore_axis_name="s"),
    scratch_types=[
        pltpu.VMEM((_C, 8, _W), jnp.float32),
        pltpu.VMEM((_C, 8, _W), jnp.float32),
        pltpu.VMEM((_C, 8, _TW), jnp.float32),
        pltpu.VMEM((_C, 8, _TW), jnp.float32),
        pltpu.VMEM((64,), jnp.float32),
        pltpu.SemaphoreType.DMA,
    ],
    compiler_params=pltpu.CompilerParams(
        needs_layout_passes=False, use_tc_tiling_on_sc=True),
)


@jax.jit
def kernel(x, y):
    part = _sc_call(x.transpose(2, 0, 1), y.transpose(2, 0, 1))
    part = part.reshape(32, 4, 16)
    face = jnp.sum(part[:, 0, :])
    mse_sum = jnp.sum(part[:, 1, :])
    bpos_sum = jnp.sum(part[:, 2, :])
    bbg_sum = jnp.sum(part[:, 3, :])
    bg_num = _TOTAL - face
    return (1.0 + 1.0 / face) * ((0.25 * mse_sum + bpos_sum) / face) \
        + bbg_sum / bg_num


# speedup vs baseline: 7.6650x; 7.6650x over previous
import functools

import jax
import jax.numpy as jnp
from jax.experimental import pallas as pl
from jax.experimental.pallas import tpu as pltpu


def _loss_kernel(total_cells, nsteps, x_ref, y_ref, out_ref, acc_ref):
    step = pl.program_id(0)

    @pl.when(step == 0)
    def _init():
        acc_ref[0] = 0.0
        acc_ref[1] = 0.0
        acc_ref[2] = 0.0
        acc_ref[3] = 0.0

    cx = x_ref[0]
    cy = y_ref[0]
    mask = (cy > 0.5).astype(jnp.float32)

    d = x_ref[1] - y_ref[1]
    sq = d * d
    d = x_ref[2] - y_ref[2]
    sq = sq + d * d
    d = x_ref[3] - y_ref[3]
    sq = sq + d * d
    d = x_ref[4] - y_ref[4]
    sq = sq + d * d

    logp = jnp.maximum(jnp.log(cx), -100.0)
    log1mp = jnp.maximum(jnp.log(1.0 - cx), -100.0)

    acc_ref[0] += jnp.sum(mask)
    acc_ref[1] += jnp.sum(mask * sq)
    acc_ref[2] += jnp.sum(mask * (cy * logp + (1.0 - cy) * log1mp))
    acc_ref[3] += jnp.sum((mask - 1.0) * log1mp)

    @pl.when(step == nsteps - 1)
    def _finalize():
        f = acc_ref[0]
        bg_num = total_cells - f
        loss = (1.0 + 1.0 / f) * ((0.25 * acc_ref[1] - acc_ref[2]) / f)
        out_ref[0, 0] = loss + acc_ref[3] / bg_num


@jax.jit
def kernel(x, y):
    B, N, C = x.shape
    xt = x.transpose(2, 0, 1)
    yt = y.transpose(2, 0, 1)

    nsteps = 4
    rb = B // nsteps

    out = pl.pallas_call(
        functools.partial(_loss_kernel, float(B * N), nsteps),
        grid=(nsteps,),
        out_shape=jax.ShapeDtypeStruct((1, 1), jnp.float32),
        in_specs=[
            pl.BlockSpec((C, rb, N), lambda i: (0, i, 0)),
            pl.BlockSpec((C, rb, N), lambda i: (0, i, 0)),
        ],
        out_specs=pl.BlockSpec(memory_space=pltpu.SMEM),
        scratch_shapes=[pltpu.SMEM((4,), jnp.float32)],
    )(xt, yt)
    return out[0, 0]


# speedup vs baseline: 8.5815x; 1.1196x over previous
import functools

import jax
import jax.numpy as jnp
from jax.experimental import pallas as pl
from jax.experimental.pallas import tpu as pltpu


def _loss_kernel(total_cells, nsteps, x_ref, y_ref, out_ref, acc_ref):
    step = pl.program_id(0)

    @pl.when(step == 0)
    def _init():
        acc_ref[0] = 0.0
        acc_ref[1] = 0.0
        acc_ref[2] = 0.0
        acc_ref[3] = 0.0

    cx = x_ref[0]
    cy = y_ref[0]
    mask = (cy > 0.5).astype(jnp.float32)

    d = x_ref[1] - y_ref[1]
    sq = d * d
    d = x_ref[2] - y_ref[2]
    sq = sq + d * d
    d = x_ref[3] - y_ref[3]
    sq = sq + d * d
    d = x_ref[4] - y_ref[4]
    sq = sq + d * d

    logp = jnp.maximum(jnp.log(cx), -100.0)
    log1mp = jnp.maximum(jnp.log(1.0 - cx), -100.0)

    acc_ref[0] += jnp.sum(mask)
    acc_ref[1] += jnp.sum(mask * sq)
    acc_ref[2] += jnp.sum(mask * (cy * logp + (1.0 - cy) * log1mp))
    acc_ref[3] += jnp.sum((mask - 1.0) * log1mp)

    @pl.when(step == nsteps - 1)
    def _finalize():
        f = acc_ref[0]
        bg_num = total_cells - f
        loss = (1.0 + 1.0 / f) * ((0.25 * acc_ref[1] - acc_ref[2]) / f)
        out_ref[0, 0] = loss + acc_ref[3] / bg_num


@jax.jit
def kernel(x, y):
    B, N, C = x.shape
    xt = x.transpose(2, 0, 1)
    yt = y.transpose(2, 0, 1)

    nsteps = 2
    rb = B // nsteps

    out = pl.pallas_call(
        functools.partial(_loss_kernel, float(B * N), nsteps),
        grid=(nsteps,),
        out_shape=jax.ShapeDtypeStruct((1, 1), jnp.float32),
        in_specs=[
            pl.BlockSpec((C, rb, N), lambda i: (0, i, 0)),
            pl.BlockSpec((C, rb, N), lambda i: (0, i, 0)),
        ],
        out_specs=pl.BlockSpec(memory_space=pltpu.SMEM),
        scratch_shapes=[pltpu.SMEM((4,), jnp.float32)],
    )(xt, yt)
    return out[0, 0]
